# baseline (speedup 1.0000x reference)
import functools

import jax
import jax.numpy as jnp
from jax import lax
from jax.experimental import pallas as pl
from jax.experimental.pallas import tpu as pltpu
from jax.experimental.pallas import tpu_sc as plsc

H = 32
NE1 = 1537
NSP = 512
NTAB = 15 * NE1 + NSP
B, N = 16, 64
EL = B * N * N
NW = 32
EPW = EL // NW
CHUNK = 64
GROUPS = 8
NCHUNK = EPW // CHUNK


def _prep_body(enc0_ref, enc1_ref, enc2_ref, w_ref, spw_ref, out_ref):
    encs = (enc0_ref[...], enc1_ref[...], enc2_ref[...])
    for k in range(15):
        d, j = k // 3, k % 3
        t = jnp.dot(encs[j], w_ref[d], preferred_element_type=jnp.float32)
        out_ref[pl.ds(k * NE1, NE1), :] = t * (1.0 / 3.0)
    s = lax.broadcasted_iota(jnp.int32, (NSP, 1), 0)
    sp = jnp.where(s == 0, 1, s)
    sp = jnp.where(sp > 1, sp - 1, sp)
    sp = jnp.minimum(sp, 5)
    out_ref[pl.ds(15 * NE1, NSP), :] = spw_ref[...] * sp.astype(jnp.float32)


def _build_table(enc0, enc1, enc2, w5, spatial_w):
    return pl.pallas_call(
        _prep_body,
        out_shape=jax.ShapeDtypeStruct((NTAB, H), jnp.float32),
    )(enc0, enc1, enc2, w5, spatial_w)


def _sc_body(tab_hbm, fidx_hbm, out_hbm, idx_v, rows_v, acc_v, htile_v, sem):
    wid = lax.axis_index("s") * 2 + lax.axis_index("c")
    offv = lax.iota(jnp.int32, 16) * NE1
    lane = lax.iota(jnp.int32, 16)

    def chunk_body(c, carry):
        e0 = pl.multiple_of(wid * EPW + c * CHUNK, CHUNK)
        pltpu.sync_copy(fidx_hbm.at[pl.ds(pl.multiple_of(e0 // 8, 8), GROUPS)],
                        idx_v)
        for r in range(GROUPS):
            def off_body(i, u, r=r):
                sl = pl.ds(i * 16, 16)
                idx_v[r, sl] = idx_v[r, sl] + offv
                return u
            lax.fori_loop(0, 8, off_body, 0)
        copies = [
            pltpu.async_copy(tab_hbm.at[idx_v.at[r]], rows_v.at[r], sem)
            for r in range(GROUPS)
        ]
        for cp in copies:
            cp.wait()
        for r in range(GROUPS):
            def acc_body(t, u, r=r):
                b0 = t * 16
                h0 = rows_v[r, b0, pl.ds(0, 16)]
                h1 = rows_v[r, b0, pl.ds(16, 16)]
                for k in range(1, 16):
                    h0 = h0 + rows_v[r, b0 + k, pl.ds(0, 16)]
                    h1 = h1 + rows_v[r, b0 + k, pl.ds(16, 16)]
                e = r * 8 + t
                acc_v[pl.ds(e * H, 16)] = h0
                acc_v[pl.ds(e * H + 16, 16)] = h1
                return u
            lax.fori_loop(0, 8, acc_body, 0)
        def tr_body(h, u):
            for g in range(CHUNK // 16):
                v = plsc.load_gather(acc_v, [lane * H + (g * 16 * H) + h])
                htile_v[h, pl.ds(g * 16, 16)] = v
            return u
        lax.fori_loop(0, H, tr_body, 0)
        b = e0 // (N * N)
        i = (e0 % (N * N)) // N
        pltpu.sync_copy(htile_v,
                        out_hbm.at[b, :, pl.ds(pl.multiple_of(i * N, N), N)])
        return carry

    lax.fori_loop(0, NCHUNK, chunk_body, 0)


def _sc_gather(tab, fidx):
    mesh = plsc.VectorSubcoreMesh(core_axis_name="c", subcore_axis_name="s",
                                  num_cores=2, num_subcores=16)
    fn = pl.kernel(
        _sc_body,
        out_type=jax.ShapeDtypeStruct((B, H, N * N), jnp.float32),
        mesh=mesh,
        compiler_params=pltpu.CompilerParams(use_tc_tiling_on_sc=False,
                                             needs_layout_passes=False),
        scratch_types=[
            pltpu.VMEM((GROUPS, 128), jnp.int32),
            pltpu.VMEM((GROUPS, 128, H), jnp.float32),
            pltpu.VMEM((CHUNK * H,), jnp.float32),
            pltpu.VMEM((H, CHUNK), jnp.float32),
            pltpu.SemaphoreType.DMA,
        ],
    )
    return fn(tab, fidx)


def _asm_body(ab_ref, sp_ref, u_ref, virt_ref, out_ref):
    s = sp_ref[0]
    sp = jnp.where(s == 0, 1, s)
    sp = jnp.where(sp > 1, sp - 1, sp)
    sp = jnp.minimum(sp, 5)
    u = u_ref[0] * (1.0 / sp.astype(jnp.float32))
    ab = ab_ref[0]
    vb = virt_ref[...].reshape(H, 1)
    out_ref[0, :, 0, :] = ab[0:1, :] + vb
    out_ref[0, :, 1:, 0:1] = ab[1:, 0:1][None, :, :] + vb.reshape(H, 1, 1)
    for x in range(1, N + 1):
        row = u[:, (x - 1) * N:x * N] + ab[x:x + 1, 1:N + 1]
        out_ref[0, :, x, pl.ds(1, N)] = row


def _assemble(attn_bias, spatial_flat, unscaled, virt_w):
    return pl.pallas_call(
        _asm_body,
        grid=(B,),
        in_specs=[
            pl.BlockSpec((1, N + 1, N + 1), lambda b: (b, 0, 0)),
            pl.BlockSpec((1, 1, N * N), lambda b: (b, 0, 0)),
            pl.BlockSpec((1, H, N * N), lambda b: (b, 0, 0)),
            pl.BlockSpec((1, H), lambda b: (0, 0)),
        ],
        out_specs=pl.BlockSpec((1, H, N + 1, N + 1), lambda b: (b, 0, 0, 0)),
        out_shape=jax.ShapeDtypeStruct((B, H, N + 1, N + 1), jnp.float32),
    )(attn_bias, spatial_flat, unscaled, virt_w)


def kernel(attn_bias, spatial_pos, edge_input, edge_enc0, edge_enc1,
           edge_enc2, edge_dis_w, spatial_w, virt_w):
    spi = spatial_pos.astype(jnp.int32)
    ei = edge_input.astype(jnp.int32).reshape(EL, 15)
    fidx = jnp.concatenate([ei, spi.reshape(EL, 1)], axis=1)
    fidx = fidx.reshape(EL // 8, 128)
    w5 = edge_dis_w.reshape(-1)[: 5 * H * H].reshape(5, H, H)

    tab = _build_table(edge_enc0, edge_enc1, edge_enc2, w5, spatial_w)
    unscaled = _sc_gather(tab, fidx)
    return _assemble(attn_bias, spi.reshape(B, 1, N * N), unscaled, virt_w)


# speedup vs baseline: 1.2493x; 1.2493x over previous
import functools

import jax
import jax.numpy as jnp
from jax import lax
from jax.experimental import pallas as pl
from jax.experimental.pallas import tpu as pltpu
from jax.experimental.pallas import tpu_sc as plsc

H = 32
NE1 = 1537
NSP = 512
NTAB = 15 * NE1 + NSP
B, N = 16, 64
EL = B * N * N
NW = 32
EPW = EL // NW
CHUNK = 64
GROUPS = 8
NCHUNK = EPW // CHUNK


def _prep_body(enc0_ref, enc1_ref, enc2_ref, w_ref, spw_ref, out_ref):
    encs = (enc0_ref[...], enc1_ref[...], enc2_ref[...])
    for k in range(15):
        d, j = k // 3, k % 3
        t = jnp.dot(encs[j], w_ref[d], preferred_element_type=jnp.float32)
        out_ref[pl.ds(k * NE1, NE1), :] = t * (1.0 / 3.0)
    s = lax.broadcasted_iota(jnp.int32, (NSP, 1), 0)
    sp = jnp.where(s == 0, 1, s)
    sp = jnp.where(sp > 1, sp - 1, sp)
    sp = jnp.minimum(sp, 5)
    out_ref[pl.ds(15 * NE1, NSP), :] = spw_ref[...] * sp.astype(jnp.float32)


def _build_table(enc0, enc1, enc2, w5, spatial_w):
    return pl.pallas_call(
        _prep_body,
        out_shape=jax.ShapeDtypeStruct((NTAB, H), jnp.float32),
    )(enc0, enc1, enc2, w5, spatial_w)


def _sc_body(tab_hbm, fidx_hbm, out_hbm, idx_v, rows_v, acc_v, htile_v, sem):
    wid = lax.axis_index("s") * 2 + lax.axis_index("c")
    offv = lax.iota(jnp.int32, 16) * NE1
    lane = lax.iota(jnp.int32, 16)

    def chunk_body(c, carry):
        e0 = pl.multiple_of(wid * EPW + c * CHUNK, CHUNK)
        pltpu.sync_copy(fidx_hbm.at[pl.ds(pl.multiple_of(e0 // 8, 8), GROUPS)],
                        idx_v)
        for r in range(GROUPS):
            def off_body(i, u, r=r):
                sl = pl.ds(i * 16, 16)
                idx_v[r, sl] = idx_v[r, sl] + offv
                return u
            lax.fori_loop(0, 8, off_body, 0)
        copies = [
            pltpu.async_copy(tab_hbm.at[idx_v.at[r]], rows_v.at[r], sem)
            for r in range(GROUPS)
        ]
        for cp in copies:
            cp.wait()
        for r in range(GROUPS):
            def acc_body(t, u, r=r):
                b0 = t * 16
                h0 = rows_v[r, b0, pl.ds(0, 16)]
                h1 = rows_v[r, b0, pl.ds(16, 16)]
                for k in range(1, 16):
                    h0 = h0 + rows_v[r, b0 + k, pl.ds(0, 16)]
                    h1 = h1 + rows_v[r, b0 + k, pl.ds(16, 16)]
                e = r * 8 + t
                acc_v[pl.ds(e * H, 16)] = h0
                acc_v[pl.ds(e * H + 16, 16)] = h1
                return u
            lax.fori_loop(0, 8, acc_body, 0)
        def tr_body(h, u):
            for g in range(CHUNK // 16):
                v = plsc.load_gather(acc_v, [lane * H + (g * 16 * H) + h])
                htile_v[h, pl.ds(g * 16, 16)] = v
            return u
        lax.fori_loop(0, H, tr_body, 0)
        b = e0 // (N * N)
        i = (e0 % (N * N)) // N
        pltpu.sync_copy(htile_v, out_hbm.at[b, :, i, :])
        return carry

    lax.fori_loop(0, NCHUNK, chunk_body, 0)


def _sc_gather(tab, fidx):
    mesh = plsc.VectorSubcoreMesh(core_axis_name="c", subcore_axis_name="s",
                                  num_cores=2, num_subcores=16)
    fn = pl.kernel(
        _sc_body,
        out_type=jax.ShapeDtypeStruct((B, H, N, N), jnp.float32),
        mesh=mesh,
        compiler_params=pltpu.CompilerParams(use_tc_tiling_on_sc=False,
                                             needs_layout_passes=False),
        scratch_types=[
            pltpu.VMEM((GROUPS, 128), jnp.int32),
            pltpu.VMEM((GROUPS, 128, H), jnp.float32),
            pltpu.VMEM((CHUNK * H,), jnp.float32),
            pltpu.VMEM((H, CHUNK), jnp.float32),
            pltpu.SemaphoreType.DMA,
        ],
    )
    return fn(tab, fidx)


def _asm_body(ab_ref, sp_ref, u_ref, virt_ref, out_ref):
    s = sp_ref[0, 0]
    sp = jnp.where(s == 0, 1, s)
    sp = jnp.where(sp > 1, sp - 1, sp)
    sp = jnp.minimum(sp, 5)
    u = u_ref[0] * (1.0 / sp.astype(jnp.float32))[None]
    ab = ab_ref[0]
    vb = virt_ref[...].reshape(H, 1)
    out_ref[0, :, 0, :] = ab[0:1, :] + vb
    out_ref[0, :, 1:, 0:1] = ab[1:, 0:1][None, :, :] + vb.reshape(H, 1, 1)
    out_ref[0, :, 1:, 1:] = u + ab[1:, 1:][None]


def _assemble(attn_bias, spatial_sq, unscaled, virt_w):
    return pl.pallas_call(
        _asm_body,
        grid=(B,),
        in_specs=[
            pl.BlockSpec((1, N + 1, N + 1), lambda b: (b, 0, 0)),
            pl.BlockSpec((1, 1, N, N), lambda b: (b, 0, 0, 0)),
            pl.BlockSpec((1, H, N, N), lambda b: (b, 0, 0, 0)),
            pl.BlockSpec((1, H), lambda b: (0, 0)),
        ],
        out_specs=pl.BlockSpec((1, H, N + 1, N + 1), lambda b: (b, 0, 0, 0)),
        out_shape=jax.ShapeDtypeStruct((B, H, N + 1, N + 1), jnp.float32),
    )(attn_bias, spatial_sq, unscaled, virt_w)


def kernel(attn_bias, spatial_pos, edge_input, edge_enc0, edge_enc1,
           edge_enc2, edge_dis_w, spatial_w, virt_w):
    spi = spatial_pos.astype(jnp.int32)
    ei = edge_input.astype(jnp.int32).reshape(EL, 15)
    fidx = jnp.concatenate([ei, spi.reshape(EL, 1)], axis=1)
    fidx = fidx.reshape(EL // 8, 128)
    w5 = edge_dis_w.reshape(-1)[: 5 * H * H].reshape(5, H, H)

    tab = _build_table(edge_enc0, edge_enc1, edge_enc2, w5, spatial_w)
    unscaled = _sc_gather(tab, fidx)
    return _assemble(attn_bias, spi.reshape(B, 1, N, N), unscaled, virt_w)


# speedup vs baseline: 1.2555x; 1.0050x over previous
import functools

import jax
import jax.numpy as jnp
from jax import lax
from jax.experimental import pallas as pl
from jax.experimental.pallas import tpu as pltpu
from jax.experimental.pallas import tpu_sc as plsc

H = 32
NE1 = 1537
NSP = 512
NTAB = 15 * NE1 + NSP
B, N = 16, 64
EL = B * N * N
NW = 32
EPW = EL // NW
CHUNK = 64
GROUPS = 8
NCHUNK = EPW // CHUNK


def _prep_body(enc0_ref, enc1_ref, enc2_ref, w_ref, spw_ref, fidx_ref,
               out_ref, oidx_ref):
    encs = (enc0_ref[...], enc1_ref[...], enc2_ref[...])
    for k in range(15):
        d, j = k // 3, k % 3
        t = jnp.dot(encs[j], w_ref[d], preferred_element_type=jnp.float32)
        out_ref[pl.ds(k * NE1, NE1), :] = t * (1.0 / 3.0)
    s = lax.broadcasted_iota(jnp.int32, (NSP, 1), 0)
    sp = jnp.where(s == 0, 1, s)
    sp = jnp.where(sp > 1, sp - 1, sp)
    sp = jnp.minimum(sp, 5)
    out_ref[pl.ds(15 * NE1, NSP), :] = spw_ref[...] * sp.astype(jnp.float32)
    off = (lax.broadcasted_iota(jnp.int32, (1, 128), 1) % 16) * NE1
    oidx_ref[...] = fidx_ref[...] + off


def _build_table(enc0, enc1, enc2, w5, spatial_w, fidx):
    return pl.pallas_call(
        _prep_body,
        out_shape=[
            jax.ShapeDtypeStruct((NTAB, H), jnp.float32),
            jax.ShapeDtypeStruct((EL // 8, 128), jnp.int32),
        ],
    )(enc0, enc1, enc2, w5, spatial_w, fidx)


def _sc_body(tab_hbm, fidx_hbm, out_hbm, idx_v, rows_v, zeros_v, dst_v,
             acc_sh, sem):
    sid = lax.axis_index("s")
    wid = sid * 2 + lax.axis_index("c")
    a0 = pl.multiple_of(sid * CHUNK, CHUNK)
    z = jnp.zeros((16,), jnp.float32)
    zi = jnp.zeros((16,), jnp.int32)
    for r in range(GROUPS):
        for e in range(8):
            dst_v[r, pl.ds(e * 16, 16)] = zi + (sid * CHUNK + r * 8 + e)
    for e in range(CHUNK):
        zeros_v[e, pl.ds(0, 16)] = z
        zeros_v[e, pl.ds(16, 16)] = z

    def chunk_body(c, carry):
        e0 = pl.multiple_of(wid * EPW + c * CHUNK, CHUNK)
        pltpu.sync_copy(fidx_hbm.at[pl.ds(pl.multiple_of(e0 // 8, 8), GROUPS)],
                        idx_v)
        copies = [
            pltpu.async_copy(tab_hbm.at[idx_v.at[r]], rows_v.at[r], sem)
            for r in range(GROUPS)
        ]
        pltpu.sync_copy(zeros_v, acc_sh.at[pl.ds(a0, CHUNK)])
        for cp in copies:
            cp.wait()
        for r in range(GROUPS):
            pltpu.sync_copy(rows_v.at[r], acc_sh.at[dst_v.at[r]], add=True)
        pltpu.sync_copy(acc_sh.at[pl.ds(a0, CHUNK)], out_hbm.at[pl.ds(e0, CHUNK)])
        return carry

    lax.fori_loop(0, NCHUNK, chunk_body, 0)


def _sc_gather(tab, fidx):
    mesh = plsc.VectorSubcoreMesh(core_axis_name="c", subcore_axis_name="s",
                                  num_cores=2, num_subcores=16)
    fn = pl.kernel(
        _sc_body,
        out_type=jax.ShapeDtypeStruct((EL, H), jnp.float32),
        mesh=mesh,
        compiler_params=pltpu.CompilerParams(use_tc_tiling_on_sc=False,
                                             needs_layout_passes=False),
        scratch_types=[
            pltpu.VMEM((GROUPS, 128), jnp.int32),
            pltpu.VMEM((GROUPS, 128, H), jnp.float32),
            pltpu.VMEM((CHUNK, H), jnp.float32),
            pltpu.VMEM((GROUPS, 128), jnp.int32),
            pltpu.VMEM_SHARED((16 * CHUNK, H), jnp.float32),
            pltpu.SemaphoreType.DMA,
        ],
    )
    return fn(tab, fidx)


def _asm_body(ab_ref, sp_ref, u_ref, virt_ref, out_ref):
    s = sp_ref[0, 0]
    sp = jnp.where(s == 0, 1, s)
    sp = jnp.where(sp > 1, sp - 1, sp)
    sp = jnp.minimum(sp, 5)
    u = u_ref[0]
    ut = u.T.reshape(H, N, N)
    u3 = ut * (1.0 / sp.astype(jnp.float32))[None]
    ab = ab_ref[0]
    vb = virt_ref[...].reshape(H, 1)
    out_ref[0, :, 0, :] = ab[0:1, :] + vb
    out_ref[0, :, 1:, 0:1] = ab[1:, 0:1][None, :, :] + vb.reshape(H, 1, 1)
    out_ref[0, :, 1:, 1:] = u3 + ab[1:, 1:][None]


def _assemble(attn_bias, spatial_sq, unscaled, virt_w):
    return pl.pallas_call(
        _asm_body,
        grid=(B,),
        in_specs=[
            pl.BlockSpec((1, N + 1, N + 1), lambda b: (b, 0, 0)),
            pl.BlockSpec((1, 1, N, N), lambda b: (b, 0, 0, 0)),
            pl.BlockSpec((1, N * N, H), lambda b: (b, 0, 0)),
            pl.BlockSpec((1, H), lambda b: (0, 0)),
        ],
        out_specs=pl.BlockSpec((1, H, N + 1, N + 1), lambda b: (b, 0, 0, 0)),
        out_shape=jax.ShapeDtypeStruct((B, H, N + 1, N + 1), jnp.float32),
    )(attn_bias, spatial_sq, unscaled, virt_w)


def kernel(attn_bias, spatial_pos, edge_input, edge_enc0, edge_enc1,
           edge_enc2, edge_dis_w, spatial_w, virt_w):
    spi = spatial_pos.astype(jnp.int32)
    ei = edge_input.astype(jnp.int32).reshape(EL, 15)
    fidx = jnp.concatenate([ei, spi.reshape(EL, 1)], axis=1)
    fidx = fidx.reshape(EL // 8, 128)
    w5 = edge_dis_w.reshape(-1)[: 5 * H * H].reshape(5, H, H)

    tab, oidx = _build_table(edge_enc0, edge_enc1, edge_enc2, w5, spatial_w,
                             fidx)
    unscaled = _sc_gather(tab, oidx)
    return _assemble(attn_bias, spi.reshape(B, 1, N, N),
                     unscaled.reshape(B, N * N, H), virt_w)


# speedup vs baseline: 1.2788x; 1.0186x over previous
import functools

import jax
import jax.numpy as jnp
from jax import lax
from jax.experimental import pallas as pl
from jax.experimental.pallas import tpu as pltpu
from jax.experimental.pallas import tpu_sc as plsc

H = 32
NE1 = 1537
NSP = 512
NTAB = 15 * NE1 + NSP
NTAB_PAD = 23680
B, N = 16, 64
EL = B * N * N
NW = 32
EPW = EL // NW
CHUNK = 64
GROUPS = 8
NCHUNK = EPW // CHUNK


def _prep_body(enc0_ref, enc1_ref, enc2_ref, w_ref, spw_ref, fidx_ref,
               out_ref, oidx_ref):
    encs = (enc0_ref[...], enc1_ref[...], enc2_ref[...])
    for k in range(15):
        d, j = k // 3, k % 3
        t = jnp.dot(encs[j], w_ref[d], preferred_element_type=jnp.float32)
        out_ref[pl.ds(k * NE1, NE1), :] = t * (1.0 / 3.0)
    s = lax.broadcasted_iota(jnp.int32, (NSP, 1), 0)
    sp = jnp.where(s == 0, 1, s)
    sp = jnp.where(sp > 1, sp - 1, sp)
    sp = jnp.minimum(sp, 5)
    out_ref[pl.ds(15 * NE1, NSP), :] = spw_ref[...] * sp.astype(jnp.float32)
    out_ref[pl.ds(NTAB, NTAB_PAD - NTAB), :] = jnp.zeros(
        (NTAB_PAD - NTAB, H), jnp.float32)
    off = (lax.broadcasted_iota(jnp.int32, (1, 128), 1) % 16) * NE1
    oidx_ref[...] = fidx_ref[...] + off


def _build_table(enc0, enc1, enc2, w5, spatial_w, fidx):
    return pl.pallas_call(
        _prep_body,
        out_shape=[
            jax.ShapeDtypeStruct((NTAB_PAD, H), jnp.float32),
            jax.ShapeDtypeStruct((EL // 8, 128), jnp.int32),
        ],
    )(enc0, enc1, enc2, w5, spatial_w, fidx)


def _sc_body(tab_hbm, fidx_hbm, out_hbm, idx_v, rows_v, zeros_v, dst_v,
             acc_sh, tab_sh, sem):
    sid = lax.axis_index("s")
    wid = sid * 2 + lax.axis_index("c")
    a0 = pl.multiple_of(sid * CHUNK, CHUNK)
    t0 = pl.multiple_of(sid * (NTAB_PAD // 16), 8)
    pltpu.sync_copy(tab_hbm.at[pl.ds(t0, NTAB_PAD // 16)],
                    tab_sh.at[pl.ds(t0, NTAB_PAD // 16)])
    plsc.subcore_barrier()
    z = jnp.zeros((16,), jnp.float32)
    zi = jnp.zeros((16,), jnp.int32)
    for r in range(GROUPS):
        for e in range(8):
            dst_v[r, pl.ds(e * 16, 16)] = zi + (sid * CHUNK + r * 8 + e)
    for e in range(CHUNK):
        zeros_v[e, pl.ds(0, 16)] = z
        zeros_v[e, pl.ds(16, 16)] = z

    def chunk_body(c, carry):
        e0 = pl.multiple_of(wid * EPW + c * CHUNK, CHUNK)
        pltpu.sync_copy(fidx_hbm.at[pl.ds(pl.multiple_of(e0 // 8, 8), GROUPS)],
                        idx_v)
        copies = [
            pltpu.async_copy(tab_sh.at[idx_v.at[r]], rows_v.at[r], sem)
            for r in range(GROUPS)
        ]
        pltpu.sync_copy(zeros_v, acc_sh.at[pl.ds(a0, CHUNK)])
        for cp in copies:
            cp.wait()
        for r in range(GROUPS):
            pltpu.sync_copy(rows_v.at[r], acc_sh.at[dst_v.at[r]], add=True)
        pltpu.sync_copy(acc_sh.at[pl.ds(a0, CHUNK)], out_hbm.at[pl.ds(e0, CHUNK)])
        return carry

    lax.fori_loop(0, NCHUNK, chunk_body, 0)


def _sc_gather(tab, fidx):
    mesh = plsc.VectorSubcoreMesh(core_axis_name="c", subcore_axis_name="s",
                                  num_cores=2, num_subcores=16)
    fn = pl.kernel(
        _sc_body,
        out_type=jax.ShapeDtypeStruct((EL, H), jnp.float32),
        mesh=mesh,
        compiler_params=pltpu.CompilerParams(use_tc_tiling_on_sc=False,
                                             needs_layout_passes=False),
        scratch_types=[
            pltpu.VMEM((GROUPS, 128), jnp.int32),
            pltpu.VMEM((GROUPS, 128, H), jnp.float32),
            pltpu.VMEM((CHUNK, H), jnp.float32),
            pltpu.VMEM((GROUPS, 128), jnp.int32),
            pltpu.VMEM_SHARED((16 * CHUNK, H), jnp.float32),
            pltpu.VMEM_SHARED((NTAB_PAD, H), jnp.float32),
            pltpu.SemaphoreType.DMA,
        ],
    )
    return fn(tab, fidx)


def _asm_body(ab_ref, sp_ref, u_ref, virt_ref, out_ref):
    s = sp_ref[0, 0]
    sp = jnp.where(s == 0, 1, s)
    sp = jnp.where(sp > 1, sp - 1, sp)
    sp = jnp.minimum(sp, 5)
    u = u_ref[0]
    ut = u.T.reshape(H, N, N)
    u3 = ut * (1.0 / sp.astype(jnp.float32))[None]
    ab = ab_ref[0]
    vb = virt_ref[...].reshape(H, 1)
    out_ref[0, :, 0, :] = ab[0:1, :] + vb
    out_ref[0, :, 1:, 0:1] = ab[1:, 0:1][None, :, :] + vb.reshape(H, 1, 1)
    out_ref[0, :, 1:, 1:] = u3 + ab[1:, 1:][None]


def _assemble(attn_bias, spatial_sq, unscaled, virt_w):
    return pl.pallas_call(
        _asm_body,
        grid=(B,),
        in_specs=[
            pl.BlockSpec((1, N + 1, N + 1), lambda b: (b, 0, 0)),
            pl.BlockSpec((1, 1, N, N), lambda b: (b, 0, 0, 0)),
            pl.BlockSpec((1, N * N, H), lambda b: (b, 0, 0)),
            pl.BlockSpec((1, H), lambda b: (0, 0)),
        ],
        out_specs=pl.BlockSpec((1, H, N + 1, N + 1), lambda b: (b, 0, 0, 0)),
        out_shape=jax.ShapeDtypeStruct((B, H, N + 1, N + 1), jnp.float32),
    )(attn_bias, spatial_sq, unscaled, virt_w)


def kernel(attn_bias, spatial_pos, edge_input, edge_enc0, edge_enc1,
           edge_enc2, edge_dis_w, spatial_w, virt_w):
    spi = spatial_pos.astype(jnp.int32)
    ei = edge_input.astype(jnp.int32).reshape(EL, 15)
    fidx = jnp.concatenate([ei, spi.reshape(EL, 1)], axis=1)
    fidx = fidx.reshape(EL // 8, 128)
    w5 = edge_dis_w.reshape(-1)[: 5 * H * H].reshape(5, H, H)

    tab, oidx = _build_table(edge_enc0, edge_enc1, edge_enc2, w5, spatial_w,
                             fidx)
    unscaled = _sc_gather(tab, oidx)
    return _assemble(attn_bias, spi.reshape(B, 1, N, N),
                     unscaled.reshape(B, N * N, H), virt_w)


# speedup vs baseline: 1.3880x; 1.0854x over previous
import functools

import jax
import jax.numpy as jnp
from jax import lax
from jax.experimental import pallas as pl
from jax.experimental.pallas import tpu as pltpu
from jax.experimental.pallas import tpu_sc as plsc

H = 32
NE1 = 1537
NSP = 512
NTAB = 15 * NE1 + NSP
NTAB_PAD = 23680
B, N = 16, 64
EL = B * N * N
NW = 32
EPW = EL // NW
CHUNK = 128
NIDX = CHUNK * 16
NCHUNK = EPW // CHUNK


def _prep_body(enc0_ref, enc1_ref, enc2_ref, w_ref, spw_ref, fidx_ref,
               out_ref, oidx_ref):
    encs = (enc0_ref[...], enc1_ref[...], enc2_ref[...])
    for k in range(15):
        d, j = k // 3, k % 3
        t = jnp.dot(encs[j], w_ref[d], preferred_element_type=jnp.float32)
        out_ref[pl.ds(k * NE1, NE1), :] = t * (1.0 / 3.0)
    s = lax.broadcasted_iota(jnp.int32, (NSP, 1), 0)
    sp = jnp.where(s == 0, 1, s)
    sp = jnp.where(sp > 1, sp - 1, sp)
    sp = jnp.minimum(sp, 5)
    out_ref[pl.ds(15 * NE1, NSP), :] = spw_ref[...] * sp.astype(jnp.float32)
    out_ref[pl.ds(NTAB, NTAB_PAD - NTAB), :] = jnp.zeros(
        (NTAB_PAD - NTAB, H), jnp.float32)
    off = (lax.broadcasted_iota(jnp.int32, (1, 128), 1) % 16) * NE1
    oidx_ref[...] = fidx_ref[...] + off


def _build_table(enc0, enc1, enc2, w5, spatial_w, fidx):
    return pl.pallas_call(
        _prep_body,
        out_shape=[
            jax.ShapeDtypeStruct((NTAB_PAD, H), jnp.float32),
            jax.ShapeDtypeStruct((EL // 8, 128), jnp.int32),
        ],
    )(enc0, enc1, enc2, w5, spatial_w, fidx)


def _sc_body(tab_hbm, fidx_hbm, out_hbm, idx_v, rows_v, zeros_v, dst_v,
             acc_sh, tab_sh, sem):
    sid = lax.axis_index("s")
    wid = sid * 2 + lax.axis_index("c")
    a0 = pl.multiple_of(sid * CHUNK, CHUNK)
    t0 = pl.multiple_of(sid * (NTAB_PAD // 16), 8)
    pltpu.sync_copy(tab_hbm.at[pl.ds(t0, NTAB_PAD // 16)],
                    tab_sh.at[pl.ds(t0, NTAB_PAD // 16)])
    plsc.subcore_barrier()
    z = jnp.zeros((16,), jnp.float32)
    zi = jnp.zeros((16,), jnp.int32)
    for e in range(CHUNK):
        dst_v[pl.ds(e * 16, 16)] = zi + (sid * CHUNK + e)
        zeros_v[e, pl.ds(0, 16)] = z
        zeros_v[e, pl.ds(16, 16)] = z

    def chunk_body(c, carry):
        e0 = pl.multiple_of(wid * EPW + c * CHUNK, CHUNK)
        pltpu.sync_copy(fidx_hbm.at[pl.ds(pl.multiple_of(e0 * 16, NIDX), NIDX)],
                        idx_v)
        cp = pltpu.async_copy(tab_sh.at[idx_v], rows_v, sem)
        pltpu.sync_copy(zeros_v, acc_sh.at[pl.ds(a0, CHUNK)])
        cp.wait()
        pltpu.sync_copy(rows_v, acc_sh.at[dst_v], add=True)
        pltpu.sync_copy(acc_sh.at[pl.ds(a0, CHUNK)], out_hbm.at[pl.ds(e0, CHUNK)])
        return carry

    lax.fori_loop(0, NCHUNK, chunk_body, 0)


def _sc_gather(tab, fidx):
    mesh = plsc.VectorSubcoreMesh(core_axis_name="c", subcore_axis_name="s",
                                  num_cores=2, num_subcores=16)
    fn = pl.kernel(
        _sc_body,
        out_type=jax.ShapeDtypeStruct((EL, H), jnp.float32),
        mesh=mesh,
        compiler_params=pltpu.CompilerParams(use_tc_tiling_on_sc=False,
                                             needs_layout_passes=False),
        scratch_types=[
            pltpu.VMEM((NIDX,), jnp.int32),
            pltpu.VMEM((NIDX, H), jnp.float32),
            pltpu.VMEM((CHUNK, H), jnp.float32),
            pltpu.VMEM((NIDX,), jnp.int32),
            pltpu.VMEM_SHARED((16 * CHUNK, H), jnp.float32),
            pltpu.VMEM_SHARED((NTAB_PAD, H), jnp.float32),
            pltpu.SemaphoreType.DMA,
        ],
    )
    return fn(tab, fidx)


def _asm_body(ab_ref, sp_ref, u_ref, virt_ref, out_ref):
    s = sp_ref[0, 0]
    sp = jnp.where(s == 0, 1, s)
    sp = jnp.where(sp > 1, sp - 1, sp)
    sp = jnp.minimum(sp, 5)
    u = u_ref[0]
    ut = u.T.reshape(H, N, N)
    u3 = ut * (1.0 / sp.astype(jnp.float32))[None]
    ab = ab_ref[0]
    vb = virt_ref[...].reshape(H, 1)
    out_ref[0, :, 0, :] = ab[0:1, :] + vb
    out_ref[0, :, 1:, 0:1] = ab[1:, 0:1][None, :, :] + vb.reshape(H, 1, 1)
    out_ref[0, :, 1:, 1:] = u3 + ab[1:, 1:][None]


def _assemble(attn_bias, spatial_sq, unscaled, virt_w):
    return pl.pallas_call(
        _asm_body,
        grid=(B,),
        in_specs=[
            pl.BlockSpec((1, N + 1, N + 1), lambda b: (b, 0, 0)),
            pl.BlockSpec((1, 1, N, N), lambda b: (b, 0, 0, 0)),
            pl.BlockSpec((1, N * N, H), lambda b: (b, 0, 0)),
            pl.BlockSpec((1, H), lambda b: (0, 0)),
        ],
        out_specs=pl.BlockSpec((1, H, N + 1, N + 1), lambda b: (b, 0, 0, 0)),
        out_shape=jax.ShapeDtypeStruct((B, H, N + 1, N + 1), jnp.float32),
    )(attn_bias, spatial_sq, unscaled, virt_w)


def kernel(attn_bias, spatial_pos, edge_input, edge_enc0, edge_enc1,
           edge_enc2, edge_dis_w, spatial_w, virt_w):
    spi = spatial_pos.astype(jnp.int32)
    ei = edge_input.astype(jnp.int32).reshape(EL, 15)
    fidx = jnp.concatenate([ei, spi.reshape(EL, 1)], axis=1)
    fidx = fidx.reshape(EL // 8, 128)
    w5 = edge_dis_w.reshape(-1)[: 5 * H * H].reshape(5, H, H)

    tab, oidx = _build_table(edge_enc0, edge_enc1, edge_enc2, w5, spatial_w,
                             fidx)
    unscaled = _sc_gather(tab, oidx.reshape(EL * 16))
    return _assemble(attn_bias, spi.reshape(B, 1, N, N),
                     unscaled.reshape(B, N * N, H), virt_w)
